# baseline (speedup 1.0000x reference)
import functools

import jax
import jax.numpy as jnp
from jax import lax
from jax.experimental import pallas as pl
from jax.experimental.pallas import tpu as pltpu
from jax.experimental.pallas import tpu_sc as plsc

NUM_E = 8
TOPK = 2
_BT = 1024


def _scores_body(xl_ref, xr_ref, wl_ref, wr_ref, b_ref, s_ref, sel_ref):
    dn = (((1,), (1,)), ((), ()))
    logits = (
        lax.dot_general(wl_ref[...], xl_ref[...], dn,
                        preferred_element_type=jnp.float32)
        + lax.dot_general(wr_ref[...], xr_ref[...], dn,
                          preferred_element_type=jnp.float32))
    s = jax.nn.sigmoid(logits)
    s_ref[...] = s
    sel_ref[...] = s + b_ref[...]


def _tc_scores(xf, weight, bias2d, chunk, n_chunks):
    n = xf.shape[0]
    dim = xf.shape[1]
    half = dim // 2
    nc_tok = n // n_chunks
    blocks = nc_tok // _BT
    off = chunk * blocks
    out = jax.ShapeDtypeStruct((NUM_E, nc_tok), jnp.float32)
    return pl.pallas_call(
        _scores_body,
        grid=(blocks,),
        in_specs=[
            pl.BlockSpec((_BT, half), lambda i: (i + off, 0)),
            pl.BlockSpec((_BT, half), lambda i: (i + off, 1)),
            pl.BlockSpec((NUM_E, half), lambda i: (0, 0)),
            pl.BlockSpec((NUM_E, half), lambda i: (0, 1)),
            pl.BlockSpec((NUM_E, 1), lambda i: (0, 0)),
        ],
        out_specs=[
            pl.BlockSpec((NUM_E, _BT), lambda i: (0, i)),
            pl.BlockSpec((NUM_E, _BT), lambda i: (0, i)),
        ],
        out_shape=[out, out],
    )(xf, xf, weight, weight, bias2d)


def _make_sc_route(n):
    info = plsc.get_sparse_core_info()
    nc, ns, nl = info.num_cores, info.num_subcores, info.num_lanes
    nw = nc * ns
    tpw = n // nw
    ngrp = tpw // nl
    mesh = plsc.VectorSubcoreMesh(core_axis_name="c", subcore_axis_name="s")

    @functools.partial(
        pl.kernel,
        out_type=[
            jax.ShapeDtypeStruct((n,), jnp.float32),
            jax.ShapeDtypeStruct((n,), jnp.float32),
            jax.ShapeDtypeStruct((n,), jnp.int32),
            jax.ShapeDtypeStruct((n,), jnp.int32),
        ],
        mesh=mesh,
        scratch_types=[
            pltpu.VMEM((NUM_E, tpw), jnp.float32),
            pltpu.VMEM((NUM_E, tpw), jnp.float32),
            pltpu.VMEM((tpw,), jnp.float32),
            pltpu.VMEM((tpw,), jnp.float32),
            pltpu.VMEM((tpw,), jnp.int32),
            pltpu.VMEM((tpw,), jnp.int32),
            pltpu.SemaphoreType.DMA,
        ],
    )
    def route(s_hbm, sel_hbm, w1_out, w2_out, i1_out, i2_out,
              s_v, sel_v, w1_v, w2_v, i1_v, i2_v, sem):
        wid = lax.axis_index("s") * nc + lax.axis_index("c")
        base = wid * tpw
        copies = []
        for e in range(NUM_E):
            copies.append(pltpu.async_copy(
                s_hbm.at[e, pl.ds(base, tpw)], s_v.at[e], sem))
            copies.append(pltpu.async_copy(
                sel_hbm.at[e, pl.ds(base, tpw)], sel_v.at[e], sem))
        for cp in copies:
            cp.wait()

        def group(g, carry):
            off = g * nl
            s = [s_v[e, pl.ds(off, nl)] for e in range(NUM_E)]
            sel = [sel_v[e, pl.ds(off, nl)] for e in range(NUM_E)]
            m1 = sel[0]
            w1 = s[0]
            i1 = jnp.zeros((nl,), jnp.int32)
            m2 = jnp.full((nl,), -jnp.inf, jnp.float32)
            w2 = jnp.zeros((nl,), jnp.float32)
            i2 = jnp.zeros((nl,), jnp.int32)
            for e in range(1, NUM_E):
                ev = jnp.full((nl,), e, jnp.int32)
                c1 = sel[e] > m1
                c2 = sel[e] > m2
                m2 = jnp.where(c1, m1, jnp.where(c2, sel[e], m2))
                w2 = jnp.where(c1, w1, jnp.where(c2, s[e], w2))
                i2 = jnp.where(c1, i1, jnp.where(c2, ev, i2))
                m1 = jnp.where(c1, sel[e], m1)
                w1 = jnp.where(c1, s[e], w1)
                i1 = jnp.where(c1, ev, i1)
            tot = w1 + w2 + 1e-20
            w1_v[pl.ds(off, nl)] = w1 / tot
            w2_v[pl.ds(off, nl)] = w2 / tot
            i1_v[pl.ds(off, nl)] = i1
            i2_v[pl.ds(off, nl)] = i2
            return carry

        lax.fori_loop(0, ngrp, group, 0)
        outs = [
            pltpu.async_copy(w1_v, w1_out.at[pl.ds(base, tpw)], sem),
            pltpu.async_copy(w2_v, w2_out.at[pl.ds(base, tpw)], sem),
            pltpu.async_copy(i1_v, i1_out.at[pl.ds(base, tpw)], sem),
            pltpu.async_copy(i2_v, i2_out.at[pl.ds(base, tpw)], sem),
        ]
        for cp in outs:
            cp.wait()

    return route


_NCHUNKS = 1


def kernel(x, weight, bias):
    b, t, dim = x.shape
    n = b * t
    xf = x.reshape(n, dim)
    bias2d = bias.reshape(NUM_E, 1)
    route = _make_sc_route(n // _NCHUNKS)
    parts = []
    for c in range(_NCHUNKS):
        s_t, sel_t = _tc_scores(xf, weight, bias2d, c, _NCHUNKS)
        parts.append(route(s_t, sel_t))
    w1, w2, i1, i2 = (jnp.concatenate([p[j] for p in parts])
                      for j in range(4))
    weights = jnp.stack([w1, w2], axis=-1).reshape(b, t, TOPK)
    indices = jnp.stack([i1, i2], axis=-1).reshape(b, t, TOPK)
    return weights, indices


# speedup vs baseline: 1.1514x; 1.1514x over previous
import functools

import jax
import jax.numpy as jnp
from jax import lax
from jax.experimental import pallas as pl
from jax.experimental.pallas import tpu as pltpu
from jax.experimental.pallas import tpu_sc as plsc

NUM_E = 8
TOPK = 2
_BT = 4096


def _scores_body(xl_ref, xr_ref, wl_ref, wr_ref, b_ref, s_ref, sel_ref):
    dn = (((1,), (1,)), ((), ()))
    logits = (
        lax.dot_general(wl_ref[...], xl_ref[...], dn,
                        preferred_element_type=jnp.float32)
        + lax.dot_general(wr_ref[...], xr_ref[...], dn,
                          preferred_element_type=jnp.float32))
    s = jax.nn.sigmoid(logits)
    s_ref[...] = s
    sel_ref[...] = s + b_ref[...]


def _tc_scores(xf, weight, bias2d, chunk, n_chunks):
    n = xf.shape[0]
    dim = xf.shape[1]
    half = dim // 2
    nc_tok = n // n_chunks
    blocks = nc_tok // _BT
    off = chunk * blocks
    out = jax.ShapeDtypeStruct((NUM_E, nc_tok), jnp.float32)
    return pl.pallas_call(
        _scores_body,
        grid=(blocks,),
        in_specs=[
            pl.BlockSpec((_BT, half), lambda i: (i + off, 0)),
            pl.BlockSpec((_BT, half), lambda i: (i + off, 1)),
            pl.BlockSpec((NUM_E, half), lambda i: (0, 0)),
            pl.BlockSpec((NUM_E, half), lambda i: (0, 1)),
            pl.BlockSpec((NUM_E, 1), lambda i: (0, 0)),
        ],
        out_specs=[
            pl.BlockSpec((NUM_E, _BT), lambda i: (0, i)),
            pl.BlockSpec((NUM_E, _BT), lambda i: (0, i)),
        ],
        out_shape=[out, out],
    )(xf, xf, weight, weight, bias2d)


def _make_sc_route(n):
    info = plsc.get_sparse_core_info()
    nc, ns, nl = info.num_cores, info.num_subcores, info.num_lanes
    nw = nc * ns
    tpw = n // nw
    ngrp = tpw // nl
    mesh = plsc.VectorSubcoreMesh(core_axis_name="c", subcore_axis_name="s")

    @functools.partial(
        pl.kernel,
        out_type=[
            jax.ShapeDtypeStruct((n,), jnp.float32),
            jax.ShapeDtypeStruct((n,), jnp.float32),
            jax.ShapeDtypeStruct((n,), jnp.int32),
            jax.ShapeDtypeStruct((n,), jnp.int32),
        ],
        mesh=mesh,
        scratch_types=[
            pltpu.VMEM((NUM_E, tpw), jnp.float32),
            pltpu.VMEM((NUM_E, tpw), jnp.float32),
            pltpu.VMEM((tpw,), jnp.float32),
            pltpu.VMEM((tpw,), jnp.float32),
            pltpu.VMEM((tpw,), jnp.int32),
            pltpu.VMEM((tpw,), jnp.int32),
            pltpu.SemaphoreType.DMA,
        ],
    )
    def route(s_hbm, sel_hbm, w1_out, w2_out, i1_out, i2_out,
              s_v, sel_v, w1_v, w2_v, i1_v, i2_v, sem):
        wid = lax.axis_index("s") * nc + lax.axis_index("c")
        base = wid * tpw
        copies = []
        for e in range(NUM_E):
            copies.append(pltpu.async_copy(
                s_hbm.at[e, pl.ds(base, tpw)], s_v.at[e], sem))
            copies.append(pltpu.async_copy(
                sel_hbm.at[e, pl.ds(base, tpw)], sel_v.at[e], sem))
        for cp in copies:
            cp.wait()

        def group(g, carry):
            off = g * nl
            s = [s_v[e, pl.ds(off, nl)] for e in range(NUM_E)]
            sel = [sel_v[e, pl.ds(off, nl)] for e in range(NUM_E)]
            m1 = sel[0]
            w1 = s[0]
            i1 = jnp.zeros((nl,), jnp.int32)
            m2 = jnp.full((nl,), -jnp.inf, jnp.float32)
            w2 = jnp.zeros((nl,), jnp.float32)
            i2 = jnp.zeros((nl,), jnp.int32)
            for e in range(1, NUM_E):
                ev = jnp.full((nl,), e, jnp.int32)
                c1 = sel[e] > m1
                c2 = sel[e] > m2
                m2 = jnp.where(c1, m1, jnp.where(c2, sel[e], m2))
                w2 = jnp.where(c1, w1, jnp.where(c2, s[e], w2))
                i2 = jnp.where(c1, i1, jnp.where(c2, ev, i2))
                m1 = jnp.where(c1, sel[e], m1)
                w1 = jnp.where(c1, s[e], w1)
                i1 = jnp.where(c1, ev, i1)
            tot = w1 + w2 + 1e-20
            w1_v[pl.ds(off, nl)] = w1 / tot
            w2_v[pl.ds(off, nl)] = w2 / tot
            i1_v[pl.ds(off, nl)] = i1
            i2_v[pl.ds(off, nl)] = i2
            return carry

        lax.fori_loop(0, ngrp, group, 0)
        outs = [
            pltpu.async_copy(w1_v, w1_out.at[pl.ds(base, tpw)], sem),
            pltpu.async_copy(w2_v, w2_out.at[pl.ds(base, tpw)], sem),
            pltpu.async_copy(i1_v, i1_out.at[pl.ds(base, tpw)], sem),
            pltpu.async_copy(i2_v, i2_out.at[pl.ds(base, tpw)], sem),
        ]
        for cp in outs:
            cp.wait()

    return route


_NCHUNKS = 1


def kernel(x, weight, bias):
    b, t, dim = x.shape
    n = b * t
    xf = x.reshape(n, dim)
    bias2d = bias.reshape(NUM_E, 1)
    route = _make_sc_route(n // _NCHUNKS)
    parts = []
    for c in range(_NCHUNKS):
        s_t, sel_t = _tc_scores(xf, weight, bias2d, c, _NCHUNKS)
        parts.append(route(s_t, sel_t))
    w1, w2, i1, i2 = (jnp.concatenate([p[j] for p in parts])
                      for j in range(4))
    weights = jnp.stack([w1, w2], axis=-1).reshape(b, t, TOPK)
    indices = jnp.stack([i1, i2], axis=-1).reshape(b, t, TOPK)
    return weights, indices
